# baseline (speedup 1.0000x reference)
import functools

import jax
import jax.numpy as jnp
from jax import lax
from jax.experimental import pallas as pl
from jax.experimental.pallas import tpu as pltpu
from jax.experimental.pallas import tpu_sc as plsc

T = 2048
D = 1024
F = 4096
E = 8
K = 2
TK = T * K

BLK = 256
G = 24
RPAD = G * BLK

NC, NS = 2, 16
NW = NC * NS
LANES = 16

_ROWS_PER_W = RPAD // NW
_GCS = 96
_TOK_PER_W = T // NW
_CCS = 32

RB = 256
EPAD = 128


def _router_body(x_ref, wr_ref, br_ref, probs_ref, tw_ref, ti_ref):
    x = x_ref[...]
    logits = jnp.dot(x, wr_ref[...], preferred_element_type=jnp.float32)
    logits = logits + br_ref[0, :][None, :]
    m = jnp.max(logits, axis=1, keepdims=True)
    ex = jnp.exp(logits - m)
    probs = ex / jnp.sum(ex, axis=1, keepdims=True)
    probs_ref[...] = probs

    iota = lax.broadcasted_iota(jnp.int32, (RB, EPAD), 1)
    m1 = jnp.max(probs, axis=1, keepdims=True)
    i1 = jnp.min(jnp.where(probs == m1, iota, EPAD), axis=1, keepdims=True)
    pm = jnp.where(iota == i1, -1.0, probs)
    m2 = jnp.max(pm, axis=1, keepdims=True)
    i2 = jnp.min(jnp.where(pm == m2, iota, EPAD), axis=1, keepdims=True)
    tw_ref[...] = jnp.where(iota == 0, m1, jnp.where(iota == 1, m2, 0.0))
    ti_ref[...] = jnp.where(iota == 0, i1, jnp.where(iota == 1, i2, 0))


def _router(x2d, wr_pad, br_pad):
    return pl.pallas_call(
        _router_body,
        grid=(T // RB,),
        in_specs=[
            pl.BlockSpec((RB, D), lambda i: (i, 0)),
            pl.BlockSpec((D, EPAD), lambda i: (0, 0)),
            pl.BlockSpec((1, EPAD), lambda i: (0, 0)),
        ],
        out_specs=[
            pl.BlockSpec((RB, EPAD), lambda i: (i, 0)),
            pl.BlockSpec((RB, EPAD), lambda i: (i, 0)),
            pl.BlockSpec((RB, EPAD), lambda i: (i, 0)),
        ],
        out_shape=[
            jax.ShapeDtypeStruct((T, EPAD), jnp.float32),
            jax.ShapeDtypeStruct((T, EPAD), jnp.float32),
            jax.ShapeDtypeStruct((T, EPAD), jnp.int32),
        ],
    )(x2d, wr_pad, br_pad)


def _sc_gather(x2d, src_tok):
    mesh = plsc.VectorSubcoreMesh(
        core_axis_name="c", subcore_axis_name="s", num_cores=NC, num_subcores=NS)

    @functools.partial(
        pl.kernel,
        out_type=jax.ShapeDtypeStruct((RPAD, D), jnp.float32),
        mesh=mesh,
        scratch_types=[
            pltpu.VMEM((_ROWS_PER_W // _GCS, _GCS), jnp.int32),
            pltpu.VMEM((_GCS, D), jnp.float32),
            pltpu.SemaphoreType.DMA,
        ],
    )
    def k(x_hbm, idx_hbm, out_hbm, idx_v, rows_v, sem):
        wid = lax.axis_index("s") * NC + lax.axis_index("c")
        base = wid * _ROWS_PER_W
        for c in range(_ROWS_PER_W // _GCS):
            off = base + c * _GCS
            pltpu.sync_copy(idx_hbm.at[pl.ds(off, _GCS)], idx_v.at[c])
            pltpu.async_copy(x_hbm.at[idx_v.at[c]], rows_v, sem).wait()
            pltpu.sync_copy(rows_v, out_hbm.at[pl.ds(off, _GCS)])

    return k(x2d, src_tok)


_FT = 2048


def _ffn_body(eid_s, vb_s, xs_ref, w1_ref, b1_ref, w2_ref, b2_ref, ws_ref,
              ys_ref):
    g = pl.program_id(0)

    @pl.when(vb_s[g] != 0)
    def _():
        x = xs_ref[...].astype(jnp.bfloat16)
        acc = jnp.zeros((BLK, D), jnp.float32)
        for fc in range(F // _FT):
            sl = slice(fc * _FT, (fc + 1) * _FT)
            h = jnp.dot(x, w1_ref[0, :, sl], preferred_element_type=jnp.float32)
            h = jnp.maximum(h + b1_ref[0, 0, sl][None, :], 0.0)
            acc = acc + jnp.dot(h.astype(jnp.bfloat16), w2_ref[0, sl, :],
                                preferred_element_type=jnp.float32)
        ys_ref[...] = (acc + b2_ref[0, 0, :][None, :]) * ws_ref[...]


def _ffn(eid, vb, xs, W1, b1, W2, b2, ws2d):
    grid_spec = pltpu.PrefetchScalarGridSpec(
        num_scalar_prefetch=2,
        grid=(G,),
        in_specs=[
            pl.BlockSpec((BLK, D), lambda g, eid, vb: (g, 0)),
            pl.BlockSpec((1, D, F), lambda g, eid, vb: (eid[g], 0, 0)),
            pl.BlockSpec((1, 1, F), lambda g, eid, vb: (eid[g], 0, 0)),
            pl.BlockSpec((1, F, D), lambda g, eid, vb: (eid[g], 0, 0)),
            pl.BlockSpec((1, 1, D), lambda g, eid, vb: (eid[g], 0, 0)),
            pl.BlockSpec((BLK, 1), lambda g, eid, vb: (g, 0)),
        ],
        out_specs=pl.BlockSpec((BLK, D), lambda g, eid, vb: (g, 0)),
    )
    return pl.pallas_call(
        _ffn_body,
        grid_spec=grid_spec,
        out_shape=jax.ShapeDtypeStruct((RPAD, D), jnp.float32),
        compiler_params=pltpu.CompilerParams(
            vmem_limit_bytes=128 * 1024 * 1024),
    )(eid, vb, xs, W1, b1, W2, b2, ws2d)


def _sc_combine(ys, p0, p1):
    mesh = plsc.VectorSubcoreMesh(
        core_axis_name="c", subcore_axis_name="s", num_cores=NC, num_subcores=NS)

    @functools.partial(
        pl.kernel,
        out_type=jax.ShapeDtypeStruct((T, D), jnp.float32),
        mesh=mesh,
        scratch_types=[
            pltpu.VMEM((_TOK_PER_W // _CCS, _CCS), jnp.int32),
            pltpu.VMEM((_TOK_PER_W // _CCS, _CCS), jnp.int32),
            pltpu.VMEM((_CCS, D), jnp.float32),
            pltpu.VMEM((_CCS, D), jnp.float32),
            pltpu.SemaphoreType.DMA,
        ],
    )
    def k(ys_hbm, p0_hbm, p1_hbm, out_hbm, i0_v, i1_v, buf0, buf1, sem):
        wid = lax.axis_index("s") * NC + lax.axis_index("c")
        base = wid * _TOK_PER_W
        for c in range(_TOK_PER_W // _CCS):
            off = base + c * _CCS
            pltpu.sync_copy(p0_hbm.at[pl.ds(off, _CCS)], i0_v.at[c])
            pltpu.sync_copy(p1_hbm.at[pl.ds(off, _CCS)], i1_v.at[c])
            pltpu.async_copy(ys_hbm.at[i0_v.at[c]], buf0, sem).wait()
            pltpu.async_copy(ys_hbm.at[i1_v.at[c]], buf1, sem).wait()

            def row(rr, carry):
                for jj in range(D // LANES):
                    sl = pl.ds(jj * LANES, LANES)
                    buf0[rr, sl] = buf0[rr, sl] + buf1[rr, sl]
                return carry

            lax.fori_loop(0, _CCS, row, 0)
            pltpu.sync_copy(buf0, out_hbm.at[pl.ds(off, _CCS)])

    return k(ys, p0, p1)


def kernel(x, Wr, br, W1, b1, W2, b2):
    b, s, d = x.shape
    x2d = x.reshape(T, D)

    wr_pad = jnp.zeros((D, EPAD), jnp.float32).at[:, :E].set(Wr)
    br_pad = jnp.full((1, EPAD), -1e30, jnp.float32).at[0, :E].set(br)

    probs_p, tw_p, ti_p = _router(x2d, wr_pad, br_pad)
    router_probs = probs_p[:, :E]
    tw = tw_p[:, :K]
    ti = ti_p[:, :K]

    keys = ti.reshape(TK)
    order = jnp.argsort(keys, stable=True).astype(jnp.int32)
    counts = jnp.bincount(keys, length=E).astype(jnp.int32)
    csum = jnp.cumsum(counts)
    raw_off = csum - counts
    cap = ((counts + BLK - 1) // BLK) * BLK
    ccap = jnp.cumsum(cap)
    al_off = ccap - cap
    used = ccap[-1]

    r = jnp.arange(RPAD, dtype=jnp.int32)
    e_r = jnp.minimum(jnp.searchsorted(ccap, r, side="right"), E - 1)
    j = r - al_off[e_r]
    vrow = (j < counts[e_r]) & (r < used)
    sidx = jnp.clip(raw_off[e_r] + j, 0, TK - 1)
    slot = order[sidx]
    src_tok = jnp.where(vrow, slot // K, 0).astype(jnp.int32)
    ws_row = jnp.where(vrow, tw.reshape(TK)[slot], 0.0)

    gb = jnp.arange(G, dtype=jnp.int32) * BLK
    eid = jnp.minimum(
        jnp.searchsorted(ccap, jnp.minimum(gb, used - 1), side="right"),
        E - 1).astype(jnp.int32)
    vb = (gb < used).astype(jnp.int32)

    pos = jnp.zeros((TK,), jnp.int32).at[order].set(
        jnp.arange(TK, dtype=jnp.int32))
    ppos = (pos + (al_off - raw_off)[keys]).astype(jnp.int32)
    p0 = ppos.reshape(T, K)[:, 0]
    p1 = ppos.reshape(T, K)[:, 1]

    xs = _sc_gather(x2d, src_tok)
    ys = _ffn(eid, vb, xs, W1.astype(jnp.bfloat16), b1.reshape(E, 1, F),
              W2.astype(jnp.bfloat16), b2.reshape(E, 1, D),
              ws_row.reshape(RPAD, 1))
    out2d = _sc_combine(ys, p0, p1)

    return out2d.reshape(b, s, d), router_probs


# speedup vs baseline: 1.0042x; 1.0042x over previous
import functools

import jax
import jax.numpy as jnp
from jax import lax
from jax.experimental import pallas as pl
from jax.experimental.pallas import tpu as pltpu
from jax.experimental.pallas import tpu_sc as plsc

T = 2048
D = 1024
F = 4096
E = 8
K = 2
TK = T * K

BLK = 256
G = 24
RPAD = G * BLK

NC, NS = 2, 16
NW = NC * NS
LANES = 16

_ROWS_PER_W = RPAD // NW
_GCS = 48
_GNC = _ROWS_PER_W // _GCS
_TOK_PER_W = T // NW
_CCS = 16
_CNC = _TOK_PER_W // _CCS

RB = 256
EPAD = 128


def _router_body(x_ref, wr_ref, br_ref, probs_ref, tw_ref, ti_ref):
    x = x_ref[...]
    logits = jnp.dot(x, wr_ref[...], preferred_element_type=jnp.float32)
    logits = logits + br_ref[0, :][None, :]
    m = jnp.max(logits, axis=1, keepdims=True)
    ex = jnp.exp(logits - m)
    probs = ex / jnp.sum(ex, axis=1, keepdims=True)
    probs_ref[...] = probs

    iota = lax.broadcasted_iota(jnp.int32, (RB, EPAD), 1)
    m1 = jnp.max(probs, axis=1, keepdims=True)
    i1 = jnp.min(jnp.where(probs == m1, iota, EPAD), axis=1, keepdims=True)
    pm = jnp.where(iota == i1, -1.0, probs)
    m2 = jnp.max(pm, axis=1, keepdims=True)
    i2 = jnp.min(jnp.where(pm == m2, iota, EPAD), axis=1, keepdims=True)
    tw_ref[...] = jnp.where(iota == 0, m1, jnp.where(iota == 1, m2, 0.0))
    ti_ref[...] = jnp.where(iota == 0, i1, jnp.where(iota == 1, i2, 0))


def _router(x2d, wr_pad, br_pad):
    return pl.pallas_call(
        _router_body,
        grid=(T // RB,),
        in_specs=[
            pl.BlockSpec((RB, D), lambda i: (i, 0)),
            pl.BlockSpec((D, EPAD), lambda i: (0, 0)),
            pl.BlockSpec((1, EPAD), lambda i: (0, 0)),
        ],
        out_specs=[
            pl.BlockSpec((RB, EPAD), lambda i: (i, 0)),
            pl.BlockSpec((RB, EPAD), lambda i: (i, 0)),
            pl.BlockSpec((RB, EPAD), lambda i: (i, 0)),
        ],
        out_shape=[
            jax.ShapeDtypeStruct((T, EPAD), jnp.float32),
            jax.ShapeDtypeStruct((T, EPAD), jnp.float32),
            jax.ShapeDtypeStruct((T, EPAD), jnp.int32),
        ],
    )(x2d, wr_pad, br_pad)


def _sc_gather(x2d, src_tok):
    mesh = plsc.VectorSubcoreMesh(
        core_axis_name="c", subcore_axis_name="s", num_cores=NC, num_subcores=NS)

    @functools.partial(
        pl.kernel,
        out_type=jax.ShapeDtypeStruct((RPAD, D), jnp.float32),
        mesh=mesh,
        scratch_types=[
            pltpu.VMEM((_GNC, _GCS), jnp.int32),
            pltpu.VMEM((_GCS, D), jnp.float32),
            pltpu.VMEM((_GCS, D), jnp.float32),
            pltpu.SemaphoreType.DMA,
            pltpu.SemaphoreType.DMA,
            pltpu.SemaphoreType.DMA,
            pltpu.SemaphoreType.DMA,
        ],
    )
    def k(x_hbm, idx_hbm, out_hbm, idx_v, rows_a, rows_b, sga, sgb, ssa, ssb):
        wid = lax.axis_index("s") * NC + lax.axis_index("c")
        base = wid * _ROWS_PER_W
        bufs = (rows_a, rows_b)
        gsem = (sga, sgb)
        ssem = (ssa, ssb)
        for c in range(_GNC):
            pltpu.sync_copy(idx_hbm.at[pl.ds(base + c * _GCS, _GCS)],
                            idx_v.at[c])
        g = [None] * _GNC
        s = [None] * _GNC
        g[0] = pltpu.async_copy(x_hbm.at[idx_v.at[0]], bufs[0], gsem[0])
        g[1] = pltpu.async_copy(x_hbm.at[idx_v.at[1]], bufs[1], gsem[1])
        for c in range(_GNC):
            b = c % 2
            g[c].wait()
            s[c] = pltpu.async_copy(
                bufs[b], out_hbm.at[pl.ds(base + c * _GCS, _GCS)], ssem[b])
            if c + 2 < _GNC:
                s[c].wait()
                g[c + 2] = pltpu.async_copy(
                    x_hbm.at[idx_v.at[c + 2]], bufs[b], gsem[b])
                s[c] = None
        for c in range(_GNC):
            if s[c] is not None:
                s[c].wait()

    return k(x2d, src_tok)


_FT = 2048


def _ffn_body(eid_s, vb_s, xs_ref, w1_ref, b1_ref, w2_ref, b2_ref, ws_ref,
              ys_ref):
    g = pl.program_id(0)

    @pl.when(vb_s[g] != 0)
    def _():
        x = xs_ref[...].astype(jnp.bfloat16)
        acc = jnp.zeros((BLK, D), jnp.float32)
        for fc in range(F // _FT):
            sl = slice(fc * _FT, (fc + 1) * _FT)
            h = jnp.dot(x, w1_ref[0, :, sl], preferred_element_type=jnp.float32)
            h = jnp.maximum(h + b1_ref[0, 0, sl][None, :], 0.0)
            acc = acc + jnp.dot(h.astype(jnp.bfloat16), w2_ref[0, sl, :],
                                preferred_element_type=jnp.float32)
        ys_ref[...] = (acc + b2_ref[0, 0, :][None, :]) * ws_ref[...]


def _ffn(eid, vb, xs, W1, b1, W2, b2, ws2d):
    grid_spec = pltpu.PrefetchScalarGridSpec(
        num_scalar_prefetch=2,
        grid=(G,),
        in_specs=[
            pl.BlockSpec((BLK, D), lambda g, eid, vb: (g, 0)),
            pl.BlockSpec((1, D, F), lambda g, eid, vb: (eid[g], 0, 0)),
            pl.BlockSpec((1, 1, F), lambda g, eid, vb: (eid[g], 0, 0)),
            pl.BlockSpec((1, F, D), lambda g, eid, vb: (eid[g], 0, 0)),
            pl.BlockSpec((1, 1, D), lambda g, eid, vb: (eid[g], 0, 0)),
            pl.BlockSpec((BLK, 1), lambda g, eid, vb: (g, 0)),
        ],
        out_specs=pl.BlockSpec((BLK, D), lambda g, eid, vb: (g, 0)),
    )
    return pl.pallas_call(
        _ffn_body,
        grid_spec=grid_spec,
        out_shape=jax.ShapeDtypeStruct((RPAD, D), jnp.float32),
        compiler_params=pltpu.CompilerParams(
            vmem_limit_bytes=128 * 1024 * 1024),
    )(eid, vb, xs, W1, b1, W2, b2, ws2d)


def _sc_combine(ys, p0, p1):
    mesh = plsc.VectorSubcoreMesh(
        core_axis_name="c", subcore_axis_name="s", num_cores=NC, num_subcores=NS)

    @functools.partial(
        pl.kernel,
        out_type=jax.ShapeDtypeStruct((T, D), jnp.float32),
        mesh=mesh,
        scratch_types=[
            pltpu.VMEM((_CNC, _CCS), jnp.int32),
            pltpu.VMEM((_CNC, _CCS), jnp.int32),
            pltpu.VMEM((_CCS, D), jnp.float32),
            pltpu.VMEM((_CCS, D), jnp.float32),
            pltpu.VMEM((_CCS, D), jnp.float32),
            pltpu.VMEM((_CCS, D), jnp.float32),
            pltpu.SemaphoreType.DMA,
            pltpu.SemaphoreType.DMA,
            pltpu.SemaphoreType.DMA,
            pltpu.SemaphoreType.DMA,
        ],
    )
    def k(ys_hbm, p0_hbm, p1_hbm, out_hbm, i0_v, i1_v,
          b0a, b1a, b0b, b1b, sga, sgb, ssa, ssb):
        wid = lax.axis_index("s") * NC + lax.axis_index("c")
        base = wid * _TOK_PER_W
        bufs = ((b0a, b1a), (b0b, b1b))
        gsem = (sga, sgb)
        ssem = (ssa, ssb)
        for c in range(_CNC):
            off = base + c * _CCS
            pltpu.sync_copy(p0_hbm.at[pl.ds(off, _CCS)], i0_v.at[c])
            pltpu.sync_copy(p1_hbm.at[pl.ds(off, _CCS)], i1_v.at[c])

        def gpair(c):
            b = c % 2
            h0 = pltpu.async_copy(ys_hbm.at[i0_v.at[c]], bufs[b][0], gsem[b])
            h1 = pltpu.async_copy(ys_hbm.at[i1_v.at[c]], bufs[b][1], gsem[b])
            return (h0, h1)

        g = [None] * _CNC
        s = [None] * _CNC
        g[0] = gpair(0)
        g[1] = gpair(1)
        for c in range(_CNC):
            b = c % 2
            g[c][0].wait()
            g[c][1].wait()
            b0, b1 = bufs[b]

            def row(rr, carry):
                for jj in range(D // LANES):
                    sl = pl.ds(jj * LANES, LANES)
                    b0[rr, sl] = b0[rr, sl] + b1[rr, sl]
                return carry

            lax.fori_loop(0, _CCS, row, 0)
            s[c] = pltpu.async_copy(
                b0, out_hbm.at[pl.ds(base + c * _CCS, _CCS)], ssem[b])
            if c + 2 < _CNC:
                s[c].wait()
                g[c + 2] = gpair(c + 2)
                s[c] = None
        for c in range(_CNC):
            if s[c] is not None:
                s[c].wait()

    return k(ys, p0, p1)


def kernel(x, Wr, br, W1, b1, W2, b2):
    b, s, d = x.shape
    x2d = x.reshape(T, D)

    wr_pad = jnp.zeros((D, EPAD), jnp.float32).at[:, :E].set(Wr)
    br_pad = jnp.full((1, EPAD), -1e30, jnp.float32).at[0, :E].set(br)

    probs_p, tw_p, ti_p = _router(x2d, wr_pad, br_pad)
    router_probs = probs_p[:, :E]
    tw = tw_p[:, :K]
    ti = ti_p[:, :K]

    keys = ti.reshape(TK)
    order = jnp.argsort(keys, stable=True).astype(jnp.int32)
    counts = jnp.bincount(keys, length=E).astype(jnp.int32)
    csum = jnp.cumsum(counts)
    raw_off = csum - counts
    cap = ((counts + BLK - 1) // BLK) * BLK
    ccap = jnp.cumsum(cap)
    al_off = ccap - cap
    used = ccap[-1]

    r = jnp.arange(RPAD, dtype=jnp.int32)
    e_r = jnp.minimum(jnp.searchsorted(ccap, r, side="right"), E - 1)
    j = r - al_off[e_r]
    vrow = (j < counts[e_r]) & (r < used)
    sidx = jnp.clip(raw_off[e_r] + j, 0, TK - 1)
    slot = order[sidx]
    src_tok = jnp.where(vrow, slot // K, 0).astype(jnp.int32)
    ws_row = jnp.where(vrow, tw.reshape(TK)[slot], 0.0)

    gb = jnp.arange(G, dtype=jnp.int32) * BLK
    eid = jnp.minimum(
        jnp.searchsorted(ccap, jnp.minimum(gb, used - 1), side="right"),
        E - 1).astype(jnp.int32)
    vb = (gb < used).astype(jnp.int32)

    pos = jnp.zeros((TK,), jnp.int32).at[order].set(
        jnp.arange(TK, dtype=jnp.int32))
    ppos = (pos + (al_off - raw_off)[keys]).astype(jnp.int32)
    p0 = ppos.reshape(T, K)[:, 0]
    p1 = ppos.reshape(T, K)[:, 1]

    xs = _sc_gather(x2d, src_tok)
    ys = _ffn(eid, vb, xs, W1.astype(jnp.bfloat16), b1.reshape(E, 1, F),
              W2.astype(jnp.bfloat16), b2.reshape(E, 1, D),
              ws_row.reshape(RPAD, 1))
    out2d = _sc_combine(ys, p0, p1)

    return out2d.reshape(b, s, d), router_probs


# speedup vs baseline: 1.2938x; 1.2885x over previous
import functools

import jax
import jax.numpy as jnp
from jax import lax
from jax.experimental import pallas as pl
from jax.experimental.pallas import tpu as pltpu
from jax.experimental.pallas import tpu_sc as plsc

T = 2048
D = 1024
F = 4096
E = 8
K = 2
TK = T * K

BLK = 256
G = 24
RPAD = G * BLK

NC, NS = 2, 16
NW = NC * NS
LANES = 16

_ROWS_PER_W = RPAD // NW
_GCS = 24
_GNC = _ROWS_PER_W // _GCS
_GNB = 4
_TOK_PER_W = T // NW
_CCS = 16
_CNC = _TOK_PER_W // _CCS

RB = 256
EPAD = 128


def _router_body(x_ref, wr_ref, br_ref, probs_ref, tw_ref, ti_ref):
    x = x_ref[...]
    logits = jnp.dot(x, wr_ref[...], preferred_element_type=jnp.float32)
    logits = logits + br_ref[0, :][None, :]
    m = jnp.max(logits, axis=1, keepdims=True)
    ex = jnp.exp(logits - m)
    probs = ex / jnp.sum(ex, axis=1, keepdims=True)
    probs_ref[...] = probs

    iota = lax.broadcasted_iota(jnp.int32, (RB, EPAD), 1)
    m1 = jnp.max(probs, axis=1, keepdims=True)
    i1 = jnp.min(jnp.where(probs == m1, iota, EPAD), axis=1, keepdims=True)
    pm = jnp.where(iota == i1, -1.0, probs)
    m2 = jnp.max(pm, axis=1, keepdims=True)
    i2 = jnp.min(jnp.where(pm == m2, iota, EPAD), axis=1, keepdims=True)
    tw_ref[...] = jnp.where(iota == 0, m1, jnp.where(iota == 1, m2, 0.0))
    ti_ref[...] = jnp.where(iota == 0, i1, jnp.where(iota == 1, i2, 0))


def _router(x2d, wr_pad, br_pad):
    return pl.pallas_call(
        _router_body,
        grid=(T // RB,),
        in_specs=[
            pl.BlockSpec((RB, D), lambda i: (i, 0)),
            pl.BlockSpec((D, EPAD), lambda i: (0, 0)),
            pl.BlockSpec((1, EPAD), lambda i: (0, 0)),
        ],
        out_specs=[
            pl.BlockSpec((RB, EPAD), lambda i: (i, 0)),
            pl.BlockSpec((RB, EPAD), lambda i: (i, 0)),
            pl.BlockSpec((RB, EPAD), lambda i: (i, 0)),
        ],
        out_shape=[
            jax.ShapeDtypeStruct((T, EPAD), jnp.float32),
            jax.ShapeDtypeStruct((T, EPAD), jnp.float32),
            jax.ShapeDtypeStruct((T, EPAD), jnp.int32),
        ],
    )(x2d, wr_pad, br_pad)


def _sc_gather(x2d, src_tok):
    mesh = plsc.VectorSubcoreMesh(
        core_axis_name="c", subcore_axis_name="s", num_cores=NC, num_subcores=NS)

    @functools.partial(
        pl.kernel,
        name="sc_dispatch_gather",
        out_type=jax.ShapeDtypeStruct((RPAD, 8, D // 8), jnp.float32),
        mesh=mesh,
        scratch_types=(
            [pltpu.VMEM((_GNC, _GCS), jnp.int32)]
            + [pltpu.VMEM((_GCS, 8, D // 8), jnp.float32) for _ in range(_GNB)]
            + [pltpu.SemaphoreType.DMA for _ in range(2 * _GNB)]
        ),
    )
    def k(x_hbm, idx_hbm, out_hbm, idx_v, *rest):
        bufs = rest[:_GNB]
        gsem = rest[_GNB:2 * _GNB]
        ssem = rest[2 * _GNB:]
        wid = lax.axis_index("s") * NC + lax.axis_index("c")
        base = wid * _ROWS_PER_W
        for c in range(_GNC):
            pltpu.sync_copy(idx_hbm.at[pl.ds(base + c * _GCS, _GCS)],
                            idx_v.at[c])
        g = [None] * _GNC
        s = [None] * _GNC
        for c in range(_GNB):
            g[c] = pltpu.async_copy(x_hbm.at[idx_v.at[c]], bufs[c], gsem[c])
        for c in range(_GNC):
            b = c % _GNB
            g[c].wait()
            s[c] = pltpu.async_copy(
                bufs[b], out_hbm.at[pl.ds(base + c * _GCS, _GCS)], ssem[b])
            if c + _GNB < _GNC:
                s[c].wait()
                g[c + _GNB] = pltpu.async_copy(
                    x_hbm.at[idx_v.at[c + _GNB]], bufs[b], gsem[b])
                s[c] = None
        for c in range(_GNC):
            if s[c] is not None:
                s[c].wait()

    return k(x2d, src_tok)


_FT = 2048


def _ffn1_body(eid_s, vb_s, xs_ref, w1_ref, b1_ref, h_ref):
    g = pl.program_id(0)

    @pl.when(vb_s[g] != 0)
    def _():
        x = xs_ref[...].astype(jnp.bfloat16)
        for fc in range(F // _FT):
            sl = slice(fc * _FT, (fc + 1) * _FT)
            h = jnp.zeros((BLK, _FT), jnp.float32)
            for s in range(8):
                h = h + jnp.dot(x[:, s, :],
                                w1_ref[0, s, :, sl].astype(jnp.bfloat16),
                                preferred_element_type=jnp.float32)
            h = jnp.maximum(h + b1_ref[0, 0, sl][None, :], 0.0)
            h_ref[:, sl] = h.astype(jnp.bfloat16)


def _ffn2_body(eid_s, vb_s, h_ref, w2_ref, b2_ref, ws_ref, ys_ref):
    g = pl.program_id(0)

    @pl.when(vb_s[g] != 0)
    def _():
        acc = jnp.zeros((BLK, D), jnp.float32)
        for fc in range(F // _FT):
            sl = slice(fc * _FT, (fc + 1) * _FT)
            acc = acc + jnp.dot(h_ref[:, sl],
                                w2_ref[0, sl, :].astype(jnp.bfloat16),
                                preferred_element_type=jnp.float32)
        ys_ref[...] = (acc + b2_ref[0, 0, :][None, :]) * ws_ref[...]


def _ffn(eid, vb, xs, W1, b1, W2, b2, ws2d):
    grid_spec1 = pltpu.PrefetchScalarGridSpec(
        num_scalar_prefetch=2,
        grid=(G,),
        in_specs=[
            pl.BlockSpec((BLK, 8, D // 8), lambda g, eid, vb: (g, 0, 0)),
            pl.BlockSpec((1, 8, D // 8, F), lambda g, eid, vb: (eid[g], 0, 0, 0)),
            pl.BlockSpec((1, 1, F), lambda g, eid, vb: (eid[g], 0, 0)),
        ],
        out_specs=pl.BlockSpec((BLK, F), lambda g, eid, vb: (g, 0)),
    )
    h = pl.pallas_call(
        _ffn1_body,
        grid_spec=grid_spec1,
        out_shape=jax.ShapeDtypeStruct((RPAD, F), jnp.bfloat16),
        compiler_params=pltpu.CompilerParams(
            vmem_limit_bytes=128 * 1024 * 1024),
    )(eid, vb, xs, W1, b1)
    grid_spec2 = pltpu.PrefetchScalarGridSpec(
        num_scalar_prefetch=2,
        grid=(G,),
        in_specs=[
            pl.BlockSpec((BLK, F), lambda g, eid, vb: (g, 0)),
            pl.BlockSpec((1, F, D), lambda g, eid, vb: (eid[g], 0, 0)),
            pl.BlockSpec((1, 1, D), lambda g, eid, vb: (eid[g], 0, 0)),
            pl.BlockSpec((BLK, 1), lambda g, eid, vb: (g, 0)),
        ],
        out_specs=pl.BlockSpec((BLK, D), lambda g, eid, vb: (g, 0)),
    )
    return pl.pallas_call(
        _ffn2_body,
        grid_spec=grid_spec2,
        out_shape=jax.ShapeDtypeStruct((RPAD, D), jnp.float32),
        compiler_params=pltpu.CompilerParams(
            vmem_limit_bytes=128 * 1024 * 1024),
    )(eid, vb, h, W2, b2, ws2d)


def _sc_combine(ys, p0, p1):
    mesh = plsc.VectorSubcoreMesh(
        core_axis_name="c", subcore_axis_name="s", num_cores=NC, num_subcores=NS)

    @functools.partial(
        pl.kernel,
        name="sc_combine_gather",
        out_type=jax.ShapeDtypeStruct((T, D), jnp.float32),
        mesh=mesh,
        scratch_types=[
            pltpu.VMEM((_CNC, _CCS), jnp.int32),
            pltpu.VMEM((_CNC, _CCS), jnp.int32),
            pltpu.VMEM((_CCS, D), jnp.float32),
            pltpu.VMEM((_CCS, D), jnp.float32),
            pltpu.VMEM((_CCS, D), jnp.float32),
            pltpu.VMEM((_CCS, D), jnp.float32),
            pltpu.SemaphoreType.DMA,
            pltpu.SemaphoreType.DMA,
            pltpu.SemaphoreType.DMA,
            pltpu.SemaphoreType.DMA,
        ],
    )
    def k(ys_hbm, p0_hbm, p1_hbm, out_hbm, i0_v, i1_v,
          b0a, b1a, b0b, b1b, sga, sgb, ssa, ssb):
        wid = lax.axis_index("s") * NC + lax.axis_index("c")
        base = wid * _TOK_PER_W
        bufs = ((b0a, b1a), (b0b, b1b))
        gsem = (sga, sgb)
        ssem = (ssa, ssb)
        for c in range(_CNC):
            off = base + c * _CCS
            pltpu.sync_copy(p0_hbm.at[pl.ds(off, _CCS)], i0_v.at[c])
            pltpu.sync_copy(p1_hbm.at[pl.ds(off, _CCS)], i1_v.at[c])

        def gpair(c):
            b = c % 2
            h0 = pltpu.async_copy(ys_hbm.at[i0_v.at[c]], bufs[b][0], gsem[b])
            h1 = pltpu.async_copy(ys_hbm.at[i1_v.at[c]], bufs[b][1], gsem[b])
            return (h0, h1)

        g = [None] * _CNC
        s = [None] * _CNC
        g[0] = gpair(0)
        g[1] = gpair(1)
        for c in range(_CNC):
            b = c % 2
            g[c][0].wait()
            g[c][1].wait()
            b0, b1 = bufs[b]

            def row(rr, carry):
                for jj in range(D // LANES):
                    sl = pl.ds(jj * LANES, LANES)
                    b0[rr, sl] = b0[rr, sl] + b1[rr, sl]
                return carry

            lax.fori_loop(0, _CCS, row, 0)
            s[c] = pltpu.async_copy(
                b0, out_hbm.at[pl.ds(base + c * _CCS, _CCS)], ssem[b])
            if c + 2 < _CNC:
                s[c].wait()
                g[c + 2] = gpair(c + 2)
                s[c] = None
        for c in range(_CNC):
            if s[c] is not None:
                s[c].wait()

    return k(ys, p0, p1)


def kernel(x, Wr, br, W1, b1, W2, b2):
    b, s, d = x.shape
    x2d = x.reshape(T, D)

    wr_pad = jnp.zeros((D, EPAD), jnp.float32).at[:, :E].set(Wr)
    br_pad = jnp.full((1, EPAD), -1e30, jnp.float32).at[0, :E].set(br)

    probs_p, tw_p, ti_p = _router(x2d, wr_pad, br_pad)
    router_probs = probs_p[:, :E]
    tw = tw_p[:, :K]
    ti = ti_p[:, :K]

    keys = ti.reshape(TK)
    onehot = (keys[:, None] == jnp.arange(E, dtype=jnp.int32)[None, :])
    onehot = onehot.astype(jnp.float32)
    _CH = 128
    o3 = onehot.reshape(TK // _CH, _CH, E)
    tri = (jnp.arange(_CH)[:, None] >= jnp.arange(_CH)[None, :])
    tri = tri.astype(jnp.float32)
    cloc = jnp.einsum("lm,cme->cle", tri, o3,
                      preferred_element_type=jnp.float32)
    tot = cloc[:, -1, :]
    ntri = (jnp.arange(TK // _CH)[:, None] > jnp.arange(TK // _CH)[None, :])
    offs = jnp.dot(ntri.astype(jnp.float32), tot,
                   preferred_element_type=jnp.float32)
    csum = (cloc + offs[:, None, :]).reshape(TK, E)
    onehot = onehot.reshape(TK, E)
    counts = csum[-1].astype(jnp.int32)
    cap = ((counts + BLK - 1) // BLK) * BLK
    ccap = jnp.cumsum(cap)
    al_off = ccap - cap
    used = ccap[-1]

    rank = jnp.sum(csum * onehot, axis=1) - 1.0
    ppos = (jnp.sum(onehot * al_off[None, :].astype(jnp.float32), axis=1)
            + rank).astype(jnp.int32)
    src_tok = jnp.zeros((RPAD,), jnp.int32).at[ppos].set(
        jnp.arange(TK, dtype=jnp.int32) // K)
    ws_row = jnp.zeros((RPAD,), jnp.float32).at[ppos].set(tw.reshape(TK))

    gb = jnp.arange(G, dtype=jnp.int32) * BLK
    gbc = jnp.minimum(gb, used - 1)
    eid = jnp.sum((gbc[:, None] >= ccap[None, :]).astype(jnp.int32), axis=1)
    vb = (gb < used).astype(jnp.int32)

    p0 = ppos.reshape(T, K)[:, 0]
    p1 = ppos.reshape(T, K)[:, 1]

    xs = _sc_gather(x2d.reshape(T, 8, D // 8), src_tok)
    ys = _ffn(eid, vb, xs,
              W1.reshape(E, 8, D // 8, F), b1.reshape(E, 1, F),
              W2, b2.reshape(E, 1, D),
              ws_row.reshape(RPAD, 1))
    out2d = _sc_combine(ys, p0, p1)

    return out2d.reshape(b, s, d), router_probs
